# baseline (speedup 1.0000x reference)
import functools

import jax
import jax.numpy as jnp
from jax import lax
from jax.experimental import pallas as pl
from jax.experimental.pallas import tpu as pltpu
from jax.experimental.pallas import tpu_sc as plsc

_NC, _NS = 2, 16
_NW = _NC * _NS
_CH = 128


@functools.lru_cache(maxsize=None)
def _make_gather(B, D, dtype_name):
    dtype = jnp.dtype(dtype_name)
    b_per_w = B // _NW
    n_ch = b_per_w // _CH
    mesh = plsc.VectorSubcoreMesh(
        core_axis_name="c", subcore_axis_name="s",
        num_cores=_NC, num_subcores=_NS)
    out_sd = jax.ShapeDtypeStruct((B, D), dtype)

    @functools.partial(
        pl.kernel,
        out_type=(out_sd, out_sd),
        mesh=mesh,
        scratch_types=[
            pltpu.VMEM((2, n_ch, _CH), jnp.int32),
            pltpu.VMEM((b_per_w, D), dtype),
            pltpu.VMEM((b_per_w, D), dtype),
            pltpu.SemaphoreType.DMA,
            pltpu.SemaphoreType.DMA,
        ],
        compiler_params=pltpu.CompilerParams(use_tc_tiling_on_sc=False),
    )
    def gather_kernel(idx_hbm, users_hbm, items_hbm, u_out, i_out,
                      idx_v, urows_v, irows_v, usem, isem):
        wid = lax.axis_index("s") * _NC + lax.axis_index("c")
        base = wid * b_per_w
        pltpu.sync_copy(idx_hbm.at[wid], idx_v)
        u_copies = [
            pltpu.async_copy(users_hbm.at[idx_v.at[0, j]],
                             urows_v.at[pl.ds(j * _CH, _CH)], usem)
            for j in range(n_ch)
        ]
        i_copies = [
            pltpu.async_copy(items_hbm.at[idx_v.at[1, j]],
                             irows_v.at[pl.ds(j * _CH, _CH)], isem)
            for j in range(n_ch)
        ]
        for c in u_copies:
            c.wait()
        pltpu.sync_copy(urows_v, u_out.at[pl.ds(base, b_per_w)])
        for c in i_copies:
            c.wait()
        pltpu.sync_copy(irows_v, i_out.at[pl.ds(base, b_per_w)])

    return gather_kernel


def kernel(inputs, users_table, items_table):
    B = inputs.shape[0]
    D = users_table.shape[1]
    b_per_w = B // _NW
    n_ch = b_per_w // _CH
    idx = inputs.T.reshape(2, _NW, n_ch, _CH).transpose(1, 0, 2, 3)
    f = _make_gather(B, D, str(users_table.dtype))
    return f(idx, users_table, items_table)
